# baseline (speedup 1.0000x reference)
import functools

import jax
import jax.numpy as jnp
from jax import lax
from jax.experimental import pallas as pl
from jax.experimental.pallas import tpu as pltpu
from jax.experimental.pallas import tpu_sc as plsc

NN = 100000
EE = 3200000
NC = 2
NS = 16
NW = NC * NS
SUB = 128
CHUNK = 1024
NSUB = CHUNK // SUB
CPW = -(-EE // (NW * CHUNK))
EPW = CPW * CHUNK
E_PAD = EPW * NW
N_PAD = 100352
RPT = N_PAD // NS
CHA = 256
NSA = CHA // SUB
CPA = EPW // CHA
ZR = RPT // 16
NZQ = 16
BLK = 1024
GRID = N_PAD // BLK

_mesh = plsc.VectorSubcoreMesh(core_axis_name="c", subcore_axis_name="s")
_params = pltpu.CompilerParams(use_tc_tiling_on_sc=False)
_f32 = jnp.float32


def _zero_fill(zbuf, n):
    def z(i, _):
        zbuf[pl.ds(i * 16, 16)] = jnp.zeros((16,), _f32)
        return 0

    lax.fori_loop(0, n // 16, z, 0)


def _deg_body(dst_hbm, ew_hbm, out_hbm, acc, zbuf, dstb0, dstb1, ewb0, ewb1,
              sdstb0, sdstb1, sewb0, sewb1, isem0, isem1, ssem0, ssem1):
    cid = lax.axis_index("c")
    sid = lax.axis_index("s")
    w = sid * NC + cid
    _zero_fill(zbuf, RPT)
    pltpu.sync_copy(zbuf, acc.at[pl.ds(sid * RPT, RPT)])
    plsc.subcore_barrier()

    rbase = w * (EPW // SUB)
    banks = ((dstb0, ewb0, sdstb0, sewb0, isem0, ssem0),
             (dstb1, ewb1, sdstb1, sewb1, isem1, ssem1))

    def issue_in(k, bank):
        dstb, ewb, _, _, isem, _ = bank
        rb = rbase + k * NSUB
        pltpu.async_copy(dst_hbm.at[pl.ds(rb, NSUB)], dstb, isem)
        pltpu.async_copy(ew_hbm.at[pl.ds(rb, NSUB)], ewb, isem)

    def process(k, bank):
        dstb, ewb, sdstb, sewb, isem, ssem = bank
        rb = rbase + k * NSUB
        pltpu.make_async_copy(dst_hbm.at[pl.ds(rb, NSUB)], dstb, isem).wait()
        pltpu.make_async_copy(ew_hbm.at[pl.ds(rb, NSUB)], ewb, isem).wait()
        @pl.when(k >= 2)
        def _():
            for j in range(NSUB):
                pltpu.make_async_copy(sewb.at[j], acc.at[sdstb.at[j]],
                                      ssem).wait()
        for j in range(NSUB):
            def cp(i, _):
                sdstb[j, pl.ds(i * 16, 16)] = dstb[j, pl.ds(i * 16, 16)]
                sewb[j, pl.ds(i * 16, 16)] = ewb[j, pl.ds(i * 16, 16)]
                return 0

            lax.fori_loop(0, SUB // 16, cp, 0)
        @pl.when(k + 2 < CPW)
        def _():
            issue_in(k + 2, bank)
        for j in range(NSUB):
            pltpu.async_copy(sewb.at[j], acc.at[sdstb.at[j]], ssem, add=True)

    issue_in(0, banks[0])
    issue_in(1, banks[1])

    def step(t, _):
        process(2 * t, banks[0])
        process(2 * t + 1, banks[1])
        return 0

    lax.fori_loop(0, CPW // 2, step, 0)
    for bank in banks:
        _, _, sdstb, sewb, _, ssem = bank
        for j in range(NSUB):
            pltpu.make_async_copy(sewb.at[j], acc.at[sdstb.at[j]],
                                  ssem).wait()
    plsc.subcore_barrier()
    pltpu.sync_copy(acc.at[pl.ds(sid * RPT, RPT)],
                    out_hbm.at[pl.ds(cid * N_PAD + sid * RPT, RPT)])


_k_deg = functools.partial(
    pl.kernel,
    out_type=jax.ShapeDtypeStruct((NC * N_PAD,), _f32),
    mesh=_mesh,
    compiler_params=_params,
    scratch_types=[
        pltpu.VMEM_SHARED((N_PAD,), _f32),
        pltpu.VMEM((RPT,), _f32),
        pltpu.VMEM((NSUB, SUB), jnp.int32),
        pltpu.VMEM((NSUB, SUB), jnp.int32),
        pltpu.VMEM((NSUB, SUB), _f32),
        pltpu.VMEM((NSUB, SUB), _f32),
        pltpu.VMEM((NSUB, SUB), jnp.int32),
        pltpu.VMEM((NSUB, SUB), jnp.int32),
        pltpu.VMEM((NSUB, SUB), _f32),
        pltpu.VMEM((NSUB, SUB), _f32),
        pltpu.SemaphoreType.DMA,
        pltpu.SemaphoreType.DMA,
        pltpu.SemaphoreType.DMA,
        pltpu.SemaphoreType.DMA,
    ],
)(_deg_body)


def _agg_body(src_hbm, dst_hbm, ew_hbm, tab_hbm, out_hbm, acc, zbuf,
              srcb0, srcb1, dstb0, dstb1, ewb0, ewb1, sdstb0, sdstb1,
              rows0, rows1, isem0, isem1, gsem0, gsem1, ssem0, ssem1):
    cid = lax.axis_index("c")
    sid = lax.axis_index("s")
    w = sid * NC + cid

    def zz(r, _):
        zbuf[r, pl.ds(0, 16)] = jnp.zeros((16,), _f32)
        return 0

    lax.fori_loop(0, ZR, zz, 0)
    for q in range(NZQ):
        pltpu.sync_copy(zbuf, acc.at[pl.ds(sid * RPT + q * ZR, ZR), :])
    plsc.subcore_barrier()

    rbase = w * (EPW // SUB)
    banks = ((srcb0, dstb0, ewb0, sdstb0, rows0, isem0, gsem0, ssem0),
             (srcb1, dstb1, ewb1, sdstb1, rows1, isem1, gsem1, ssem1))

    def issue_in(k, bank):
        srcb, dstb, ewb, _, _, isem, _, _ = bank
        rb = rbase + k * NSA
        pltpu.async_copy(src_hbm.at[pl.ds(rb, NSA)], srcb, isem)
        pltpu.async_copy(dst_hbm.at[pl.ds(rb, NSA)], dstb, isem)
        pltpu.async_copy(ew_hbm.at[pl.ds(rb, NSA)], ewb, isem)

    def process(k, bank):
        srcb, dstb, ewb, sdstb, rows, isem, gsem, ssem = bank
        rb = rbase + k * NSA
        pltpu.make_async_copy(src_hbm.at[pl.ds(rb, NSA)], srcb, isem).wait()
        pltpu.make_async_copy(dst_hbm.at[pl.ds(rb, NSA)], dstb, isem).wait()
        pltpu.make_async_copy(ew_hbm.at[pl.ds(rb, NSA)], ewb, isem).wait()
        @pl.when(k >= 2)
        def _():
            for j in range(NSA):
                pltpu.make_async_copy(rows.at[pl.ds(j * SUB, SUB), :],
                                      acc.at[sdstb.at[j]], ssem).wait()
        gd = []
        for j in range(NSA):
            gd.append(pltpu.async_copy(tab_hbm.at[srcb.at[j]],
                                       rows.at[pl.ds(j * SUB, SUB), :], gsem))
        for d in gd:
            d.wait()
        for j in range(NSA):
            def m(g, _):
                ev = ewb[j, pl.ds(g * 16, 16)]
                base = j * SUB + g * 16
                for i in range(16):
                    bi = jnp.full((16,), i, jnp.int32)
                    bc = jnp.take_along_axis(ev, bi, axis=0)
                    rows[base + i, pl.ds(0, 16)] = (
                        rows[base + i, pl.ds(0, 16)] * bc)
                return 0

            lax.fori_loop(0, SUB // 16, m, 0)
        for j in range(NSA):
            def cp(i, _):
                sdstb[j, pl.ds(i * 16, 16)] = dstb[j, pl.ds(i * 16, 16)]
                return 0

            lax.fori_loop(0, SUB // 16, cp, 0)
        @pl.when(k + 2 < CPA)
        def _():
            issue_in(k + 2, bank)
        for j in range(NSA):
            pltpu.async_copy(rows.at[pl.ds(j * SUB, SUB), :],
                             acc.at[sdstb.at[j]], ssem, add=True)

    issue_in(0, banks[0])
    issue_in(1, banks[1])

    def step(t, _):
        process(2 * t, banks[0])
        process(2 * t + 1, banks[1])
        return 0

    lax.fori_loop(0, CPA // 2, step, 0)
    for bank in banks:
        _, _, _, sdstb, rows, _, _, ssem = bank
        for j in range(NSA):
            pltpu.make_async_copy(rows.at[pl.ds(j * SUB, SUB), :],
                                  acc.at[sdstb.at[j]], ssem).wait()
    plsc.subcore_barrier()
    pltpu.sync_copy(acc.at[pl.ds(sid * RPT, RPT), :],
                    out_hbm.at[pl.ds(cid * N_PAD + sid * RPT, RPT), :])


_k_agg = functools.partial(
    pl.kernel,
    out_type=jax.ShapeDtypeStruct((NC * N_PAD, 16), _f32),
    mesh=_mesh,
    compiler_params=_params,
    scratch_types=[
        pltpu.VMEM_SHARED((N_PAD, 16), _f32),
        pltpu.VMEM((ZR, 16), _f32),
        pltpu.VMEM((NSA, SUB), jnp.int32),
        pltpu.VMEM((NSA, SUB), jnp.int32),
        pltpu.VMEM((NSA, SUB), jnp.int32),
        pltpu.VMEM((NSA, SUB), jnp.int32),
        pltpu.VMEM((NSA, SUB), _f32),
        pltpu.VMEM((NSA, SUB), _f32),
        pltpu.VMEM((NSA, SUB), jnp.int32),
        pltpu.VMEM((NSA, SUB), jnp.int32),
        pltpu.VMEM((CHA, 16), _f32),
        pltpu.VMEM((CHA, 16), _f32),
        pltpu.SemaphoreType.DMA,
        pltpu.SemaphoreType.DMA,
        pltpu.SemaphoreType.DMA,
        pltpu.SemaphoreType.DMA,
        pltpu.SemaphoreType.DMA,
        pltpu.SemaphoreType.DMA,
    ],
)(_agg_body)


def _prep_body(degp_ref, x_ref, dinv_ref, xs_ref):
    d = degp_ref[0] + degp_ref[1] + 1.0
    di = lax.rsqrt(d)[:, None]
    dinv_ref[...] = di
    xs_ref[...] = x_ref[...] * di


def _k_prep(degp, x_rows):
    return pl.pallas_call(
        _prep_body,
        grid=(GRID,),
        in_specs=[
            pl.BlockSpec((NC, BLK), lambda i: (0, i)),
            pl.BlockSpec((BLK, 16), lambda i: (i, 0)),
        ],
        out_specs=[
            pl.BlockSpec((BLK, 1), lambda i: (i, 0)),
            pl.BlockSpec((BLK, 16), lambda i: (i, 0)),
        ],
        out_shape=[
            jax.ShapeDtypeStruct((N_PAD, 1), _f32),
            jax.ShapeDtypeStruct((N_PAD, 16), _f32),
        ],
    )(degp, x_rows)


def _mid_body(racc_ref, xs_ref, dinv_ref, w1_ref, b1_ref, w2_ref, hs_ref):
    di = dinv_ref[...]
    a = (racc_ref[0] + racc_ref[1] + xs_ref[...]) * di
    h = jnp.maximum(
        jnp.dot(a, w1_ref[...], preferred_element_type=_f32) + b1_ref[...],
        0.0)
    hs_ref[...] = jnp.dot(h, w2_ref[...], preferred_element_type=_f32) * di


def _k_mid(racc1, xs, dinv, w1p, b1r, w2p):
    return pl.pallas_call(
        _mid_body,
        grid=(GRID,),
        in_specs=[
            pl.BlockSpec((NC, BLK, 16), lambda i: (0, i, 0)),
            pl.BlockSpec((BLK, 16), lambda i: (i, 0)),
            pl.BlockSpec((BLK, 1), lambda i: (i, 0)),
            pl.BlockSpec((16, 16), lambda i: (0, 0)),
            pl.BlockSpec((1, 16), lambda i: (0, 0)),
            pl.BlockSpec((16, 16), lambda i: (0, 0)),
        ],
        out_specs=pl.BlockSpec((BLK, 16), lambda i: (i, 0)),
        out_shape=jax.ShapeDtypeStruct((N_PAD, 16), _f32),
    )(racc1, xs, dinv, w1p, b1r, w2p)


def _out_body(racc_ref, hs_ref, dinv_ref, b2_ref, out_ref):
    di = dinv_ref[...]
    out_ref[...] = (racc_ref[0] + racc_ref[1] + hs_ref[...]) * di + b2_ref[...]


def _k_out(racc2, hs, dinv, b2r):
    return pl.pallas_call(
        _out_body,
        grid=(GRID,),
        in_specs=[
            pl.BlockSpec((NC, BLK, 16), lambda i: (0, i, 0)),
            pl.BlockSpec((BLK, 16), lambda i: (i, 0)),
            pl.BlockSpec((BLK, 1), lambda i: (i, 0)),
            pl.BlockSpec((1, 16), lambda i: (0, 0)),
        ],
        out_specs=pl.BlockSpec((BLK, 16), lambda i: (i, 0)),
        out_shape=jax.ShapeDtypeStruct((N_PAD, 16), _f32),
    )(racc2, hs, dinv, b2r)


def kernel(x, edge_index, edge_weight, W1, b1, W2, b2):
    src = edge_index[0]
    dst = edge_index[1]
    pad = E_PAD - EE
    src_p = jnp.concatenate([src, jnp.zeros((pad,), src.dtype)])
    dst_p = jnp.concatenate([dst, jnp.full((pad,), NN, dst.dtype)])
    ew_p = jnp.concatenate([edge_weight, jnp.zeros((pad,), _f32)])
    src_r = src_p.reshape(E_PAD // SUB, SUB)
    dst_r = dst_p.reshape(E_PAD // SUB, SUB)
    ew_r = ew_p.reshape(E_PAD // SUB, SUB)

    x_rows = jnp.zeros((N_PAD, 16), _f32).at[:NN, :3].set(x)
    w1p = jnp.zeros((16, 16), _f32).at[:3, :].set(W1)
    w2p = jnp.zeros((16, 16), _f32).at[:, :7].set(W2)
    b1r = b1.reshape(1, 16)
    b2r = jnp.zeros((1, 16), _f32).at[0, :7].set(b2)

    degp = _k_deg(dst_r, ew_r).reshape(NC, N_PAD)
    dinv, xs = _k_prep(degp, x_rows)
    racc1 = _k_agg(src_r, dst_r, ew_r, xs).reshape(NC, N_PAD, 16)
    hs = _k_mid(racc1, xs, dinv, w1p, b1r, w2p)
    racc2 = _k_agg(src_r, dst_r, ew_r, hs).reshape(NC, N_PAD, 16)
    outt = _k_out(racc2, hs, dinv, b2r)
    return outt[:NN, :7]


# speedup vs baseline: 1.3030x; 1.3030x over previous
import functools

import jax
import jax.numpy as jnp
from jax import lax
from jax.experimental import pallas as pl
from jax.experimental.pallas import tpu as pltpu
from jax.experimental.pallas import tpu_sc as plsc

NN = 100000
EE = 3200000
NC = 2
NS = 16
NW = NC * NS
SUB = 128
CHUNK = 1024
NSUB = CHUNK // SUB
CPW = -(-EE // (NW * CHUNK))
EPW = CPW * CHUNK
E_PAD = EPW * NW
N_PAD = 100352
RPT = N_PAD // NS
BLK = 1024
GRID = N_PAD // BLK

_mesh = plsc.VectorSubcoreMesh(core_axis_name="c", subcore_axis_name="s")
_f32 = jnp.float32


def _zero_fill(zbuf, n):
    def z(i, _):
        zbuf[pl.ds(i * 16, 16)] = jnp.zeros((16,), _f32)
        return 0

    lax.fori_loop(0, n // 16, z, 0)


def _deg_body(dst_hbm, ew_hbm, out_hbm, acc, zbuf, dstb0, dstb1, ewb0, ewb1,
              sdstb0, sdstb1, sewb0, sewb1, isem0, isem1, ssem0, ssem1):
    cid = lax.axis_index("c")
    sid = lax.axis_index("s")
    w = sid * NC + cid
    _zero_fill(zbuf, RPT)
    pltpu.sync_copy(zbuf, acc.at[pl.ds(sid * RPT, RPT)])
    plsc.subcore_barrier()

    rbase = w * (EPW // SUB)
    banks = ((dstb0, ewb0, sdstb0, sewb0, isem0, ssem0),
             (dstb1, ewb1, sdstb1, sewb1, isem1, ssem1))

    def issue_in(k, bank):
        dstb, ewb, _, _, isem, _ = bank
        rb = rbase + k * NSUB
        pltpu.async_copy(dst_hbm.at[pl.ds(rb, NSUB)], dstb, isem)
        pltpu.async_copy(ew_hbm.at[pl.ds(rb, NSUB)], ewb, isem)

    def process(k, bank):
        dstb, ewb, sdstb, sewb, isem, ssem = bank
        rb = rbase + k * NSUB
        pltpu.make_async_copy(dst_hbm.at[pl.ds(rb, NSUB)], dstb, isem).wait()
        pltpu.make_async_copy(ew_hbm.at[pl.ds(rb, NSUB)], ewb, isem).wait()
        @pl.when(k >= 2)
        def _():
            for j in range(NSUB):
                pltpu.make_async_copy(sewb.at[j], acc.at[sdstb.at[j]],
                                      ssem).wait()
        for j in range(NSUB):
            def cp(i, _):
                sdstb[j, pl.ds(i * 16, 16)] = dstb[j, pl.ds(i * 16, 16)]
                sewb[j, pl.ds(i * 16, 16)] = ewb[j, pl.ds(i * 16, 16)]
                return 0

            lax.fori_loop(0, SUB // 16, cp, 0)
        @pl.when(k + 2 < CPW)
        def _():
            issue_in(k + 2, bank)
        for j in range(NSUB):
            pltpu.async_copy(sewb.at[j], acc.at[sdstb.at[j]], ssem, add=True)

    issue_in(0, banks[0])
    issue_in(1, banks[1])

    def step(t, _):
        process(2 * t, banks[0])
        process(2 * t + 1, banks[1])
        return 0

    lax.fori_loop(0, CPW // 2, step, 0)
    for bank in banks:
        _, _, sdstb, sewb, _, ssem = bank
        for j in range(NSUB):
            pltpu.make_async_copy(sewb.at[j], acc.at[sdstb.at[j]],
                                  ssem).wait()
    plsc.subcore_barrier()
    pltpu.sync_copy(acc.at[pl.ds(sid * RPT, RPT)],
                    out_hbm.at[pl.ds(cid * N_PAD + sid * RPT, RPT)])


_k_deg = functools.partial(
    pl.kernel,
    out_type=jax.ShapeDtypeStruct((NC * N_PAD,), _f32),
    mesh=_mesh,
    scratch_types=[
        pltpu.VMEM_SHARED((N_PAD,), _f32),
        pltpu.VMEM((RPT,), _f32),
        pltpu.VMEM((NSUB, SUB), jnp.int32),
        pltpu.VMEM((NSUB, SUB), jnp.int32),
        pltpu.VMEM((NSUB, SUB), _f32),
        pltpu.VMEM((NSUB, SUB), _f32),
        pltpu.VMEM((NSUB, SUB), jnp.int32),
        pltpu.VMEM((NSUB, SUB), jnp.int32),
        pltpu.VMEM((NSUB, SUB), _f32),
        pltpu.VMEM((NSUB, SUB), _f32),
        pltpu.SemaphoreType.DMA,
        pltpu.SemaphoreType.DMA,
        pltpu.SemaphoreType.DMA,
        pltpu.SemaphoreType.DMA,
    ],
)(_deg_body)


def _agg_body(F, src_hbm, dst_hbm, ew_hbm, *rest):
    tabs_hbm = rest[0:F]
    out_hbm = rest[F]
    scr = rest[F + 1:]
    tabs = scr[0:F]
    accs = scr[F:2 * F]
    bufs0 = scr[2 * F:3 * F]
    bufs1 = scr[3 * F:4 * F]
    (zbuf, srcb0, srcb1, dstb0, dstb1, ewb0, ewb1, sdstb0, sdstb1,
     isem0, isem1, gsem0, gsem1, g2sem0, g2sem1, ssem0, ssem1) = scr[4 * F:]
    cid = lax.axis_index("c")
    sid = lax.axis_index("s")
    w = sid * NC + cid

    for c in range(F):
        pltpu.sync_copy(tabs_hbm[c].at[pl.ds(sid * RPT, RPT)],
                        tabs[c].at[pl.ds(sid * RPT, RPT)])
    _zero_fill(zbuf, RPT)
    for c in range(F):
        pltpu.sync_copy(zbuf, accs[c].at[pl.ds(sid * RPT, RPT)])
    plsc.subcore_barrier()

    rbase = w * (EPW // SUB)
    banks = ((srcb0, dstb0, ewb0, sdstb0, bufs0, isem0, gsem0, g2sem0, ssem0),
             (srcb1, dstb1, ewb1, sdstb1, bufs1, isem1, gsem1, g2sem1, ssem1))

    def issue_in(k, bank):
        srcb, dstb, ewb, _, _, isem, _, _, _ = bank
        rb = rbase + k * NSUB
        pltpu.async_copy(src_hbm.at[pl.ds(rb, NSUB)], srcb, isem)
        pltpu.async_copy(dst_hbm.at[pl.ds(rb, NSUB)], dstb, isem)
        pltpu.async_copy(ew_hbm.at[pl.ds(rb, NSUB)], ewb, isem)

    HSUB = NSUB // 2

    def process(k, bank):
        srcb, dstb, ewb, sdstb, bufs, isem, gsem, g2sem, ssem = bank
        rb = rbase + k * NSUB
        pltpu.make_async_copy(src_hbm.at[pl.ds(rb, NSUB)], srcb, isem).wait()
        pltpu.make_async_copy(dst_hbm.at[pl.ds(rb, NSUB)], dstb, isem).wait()
        pltpu.make_async_copy(ew_hbm.at[pl.ds(rb, NSUB)], ewb, isem).wait()
        @pl.when(k >= 2)
        def _():
            for j in range(NSUB):
                for c in range(F):
                    pltpu.make_async_copy(bufs[c].at[j],
                                          accs[c].at[sdstb.at[j]],
                                          ssem).wait()
        gd = []
        g2 = []
        for j in range(NSUB):
            sem = gsem if j < HSUB else g2sem
            lst = gd if j < HSUB else g2
            for c in range(F):
                lst.append(pltpu.async_copy(tabs[c].at[srcb.at[j]],
                                            bufs[c].at[j], sem))

        def mul(j):
            def m(i, _):
                ev = ewb[j, pl.ds(i * 16, 16)]
                for c in range(F):
                    bufs[c][j, pl.ds(i * 16, 16)] = (
                        bufs[c][j, pl.ds(i * 16, 16)] * ev)
                return 0

            lax.fori_loop(0, SUB // 16, m, 0)

        for d in gd:
            d.wait()
        for j in range(HSUB):
            mul(j)
        for d in g2:
            d.wait()
        for j in range(HSUB, NSUB):
            mul(j)
        for j in range(NSUB):
            def cp(i, _):
                sdstb[j, pl.ds(i * 16, 16)] = dstb[j, pl.ds(i * 16, 16)]
                return 0

            lax.fori_loop(0, SUB // 16, cp, 0)
        @pl.when(k + 2 < CPW)
        def _():
            issue_in(k + 2, bank)
        for j in range(NSUB):
            for c in range(F):
                pltpu.async_copy(bufs[c].at[j], accs[c].at[sdstb.at[j]], ssem,
                                 add=True)

    issue_in(0, banks[0])
    issue_in(1, banks[1])

    def step(t, _):
        process(2 * t, banks[0])
        process(2 * t + 1, banks[1])
        return 0

    lax.fori_loop(0, CPW // 2, step, 0)
    for bank in banks:
        srcb, dstb, ewb, sdstb, bufs, _, _, _, ssem = bank
        for j in range(NSUB):
            for c in range(F):
                pltpu.make_async_copy(bufs[c].at[j], accs[c].at[sdstb.at[j]],
                                      ssem).wait()
    plsc.subcore_barrier()
    for c in range(F):
        pltpu.sync_copy(
            accs[c].at[pl.ds(sid * RPT, RPT)],
            out_hbm.at[pl.ds((cid * F + c) * N_PAD + sid * RPT, RPT)])


def _make_agg(F):
    return functools.partial(
        pl.kernel,
        out_type=jax.ShapeDtypeStruct((NC * F * N_PAD,), _f32),
        mesh=_mesh,
        scratch_types=(
            [pltpu.VMEM_SHARED((N_PAD,), _f32) for _ in range(2 * F)]
            + [pltpu.VMEM((NSUB, SUB), _f32) for _ in range(2 * F)]
            + [
                pltpu.VMEM((RPT,), _f32),
                pltpu.VMEM((NSUB, SUB), jnp.int32),
                pltpu.VMEM((NSUB, SUB), jnp.int32),
                pltpu.VMEM((NSUB, SUB), jnp.int32),
                pltpu.VMEM((NSUB, SUB), jnp.int32),
                pltpu.VMEM((NSUB, SUB), _f32),
                pltpu.VMEM((NSUB, SUB), _f32),
                pltpu.VMEM((NSUB, SUB), jnp.int32),
                pltpu.VMEM((NSUB, SUB), jnp.int32),
                pltpu.SemaphoreType.DMA,
                pltpu.SemaphoreType.DMA,
                pltpu.SemaphoreType.DMA,
                pltpu.SemaphoreType.DMA,
                pltpu.SemaphoreType.DMA,
                pltpu.SemaphoreType.DMA,
                pltpu.SemaphoreType.DMA,
                pltpu.SemaphoreType.DMA,
            ]
        ),
    )(functools.partial(_agg_body, F))


_k_agg3 = _make_agg(3)
_k_agg7 = _make_agg(7)


def _prep_body(degp_ref, xt_ref, dinv_ref, xs_ref):
    d = degp_ref[0] + degp_ref[1] + 1.0
    di = lax.rsqrt(d)
    dinv_ref[...] = di[None, :]
    xs_ref[...] = xt_ref[...] * di[None, :]


def _k_prep(degp, xt):
    return pl.pallas_call(
        _prep_body,
        grid=(GRID,),
        in_specs=[
            pl.BlockSpec((NC, BLK), lambda i: (0, i)),
            pl.BlockSpec((3, BLK), lambda i: (0, i)),
        ],
        out_specs=[
            pl.BlockSpec((1, BLK), lambda i: (0, i)),
            pl.BlockSpec((3, BLK), lambda i: (0, i)),
        ],
        out_shape=[
            jax.ShapeDtypeStruct((1, N_PAD), _f32),
            jax.ShapeDtypeStruct((3, N_PAD), _f32),
        ],
    )(degp, xt)


def _mid_body(racc_ref, xs_ref, dinv_ref, w1t_ref, b1_ref, w2t_ref, hs_ref):
    di = dinv_ref[...]
    a = (racc_ref[0] + racc_ref[1] + xs_ref[...]) * di
    h = jnp.maximum(
        jnp.dot(w1t_ref[...], a, preferred_element_type=_f32) + b1_ref[...],
        0.0)
    hs_ref[...] = jnp.dot(w2t_ref[...], h, preferred_element_type=_f32) * di


def _k_mid(racc1, xs, dinv, w1t, b1c, w2t):
    return pl.pallas_call(
        _mid_body,
        grid=(GRID,),
        in_specs=[
            pl.BlockSpec((NC, 3, BLK), lambda i: (0, 0, i)),
            pl.BlockSpec((3, BLK), lambda i: (0, i)),
            pl.BlockSpec((1, BLK), lambda i: (0, i)),
            pl.BlockSpec((16, 3), lambda i: (0, 0)),
            pl.BlockSpec((16, 1), lambda i: (0, 0)),
            pl.BlockSpec((7, 16), lambda i: (0, 0)),
        ],
        out_specs=pl.BlockSpec((7, BLK), lambda i: (0, i)),
        out_shape=jax.ShapeDtypeStruct((7, N_PAD), _f32),
    )(racc1, xs, dinv, w1t, b1c, w2t)


def _out_body(racc_ref, hs_ref, dinv_ref, b2_ref, out_ref):
    di = dinv_ref[...]
    out_ref[...] = (racc_ref[0] + racc_ref[1] + hs_ref[...]) * di + b2_ref[...]


def _k_out(racc2, hs, dinv, b2c):
    return pl.pallas_call(
        _out_body,
        grid=(GRID,),
        in_specs=[
            pl.BlockSpec((NC, 7, BLK), lambda i: (0, 0, i)),
            pl.BlockSpec((7, BLK), lambda i: (0, i)),
            pl.BlockSpec((1, BLK), lambda i: (0, i)),
            pl.BlockSpec((7, 1), lambda i: (0, 0)),
        ],
        out_specs=pl.BlockSpec((7, BLK), lambda i: (0, i)),
        out_shape=jax.ShapeDtypeStruct((7, N_PAD), _f32),
    )(racc2, hs, dinv, b2c)


def kernel(x, edge_index, edge_weight, W1, b1, W2, b2):
    src = edge_index[0]
    dst = edge_index[1]
    pad = E_PAD - EE
    src_p = jnp.concatenate([src, jnp.zeros((pad,), src.dtype)])
    dst_p = jnp.concatenate([dst, jnp.full((pad,), NN, dst.dtype)])
    ew_p = jnp.concatenate([edge_weight, jnp.zeros((pad,), _f32)])
    src_r = src_p.reshape(E_PAD // SUB, SUB)
    dst_r = dst_p.reshape(E_PAD // SUB, SUB)
    ew_r = ew_p.reshape(E_PAD // SUB, SUB)

    xt = jnp.zeros((3, N_PAD), _f32).at[:, :NN].set(x.T)
    w1t = W1.T
    w2t = W2.T
    b1c = b1.reshape(16, 1)
    b2c = b2.reshape(7, 1)

    degp = _k_deg(dst_r, ew_r).reshape(NC, N_PAD)
    dinv, xs = _k_prep(degp, xt)
    racc1 = _k_agg3(src_r, dst_r, ew_r,
                    *(xs[c] for c in range(3))).reshape(NC, 3, N_PAD)
    hs = _k_mid(racc1, xs, dinv, w1t, b1c, w2t)
    racc2 = _k_agg7(src_r, dst_r, ew_r,
                    *(hs[c] for c in range(7))).reshape(NC, 7, N_PAD)
    outt = _k_out(racc2, hs, dinv, b2c)
    return outt[:, :NN].T
